# Initial kernel scaffold; baseline (speedup 1.0000x reference)
import jax
import jax.numpy as jnp
from jax.experimental import pallas as pl


def kernel(x, gate_w, w_gate, w_up, w_down):
    raise NotImplementedError("write your pallas kernel here")


# speedup vs baseline: 1.2940x; 1.2940x over previous
import jax
import jax.numpy as jnp
from jax.experimental import pallas as pl

EMBED_DIM = 1024
NUM_EXPERTS = 8
HIDDEN_DIM = 1792


def _moe_kernel(x_ref, coef_ref, wg_ref, wu_ref, wd_ref, out_ref):
    e = pl.program_id(1)
    xb = x_ref[...]
    g = jax.lax.dot_general(
        xb, wg_ref[0],
        dimension_numbers=(((1,), (1,)), ((), ())),
        preferred_element_type=jnp.float32)
    u = jax.lax.dot_general(
        xb, wu_ref[0],
        dimension_numbers=(((1,), (1,)), ((), ())),
        preferred_element_type=jnp.float32)
    h = (g * jax.nn.sigmoid(g) * u).astype(jnp.bfloat16)
    y = jax.lax.dot_general(
        h, wd_ref[0],
        dimension_numbers=(((1,), (1,)), ((), ())),
        preferred_element_type=jnp.float32)
    ei = jax.lax.broadcasted_iota(jnp.int32, coef_ref.shape, 1)
    coef_col = jnp.sum(coef_ref[...] * (ei == e).astype(jnp.float32),
                       axis=1, keepdims=True)
    contrib = y * coef_col

    @pl.when(e == 0)
    def _():
        out_ref[...] = contrib

    @pl.when(e != 0)
    def _():
        out_ref[...] += contrib


@jax.jit
def kernel(x, gate_w, w_gate, w_up, w_down):
    Bb, Ss, D = x.shape
    N = Bb * Ss
    E = NUM_EXPERTS
    H = HIDDEN_DIM
    x_flat = x.reshape(N, D)
    logits = (x_flat.astype(jnp.bfloat16)
              @ gate_w.astype(jnp.bfloat16).T).astype(jnp.float32)
    topk_w, topk_i = jax.lax.top_k(logits, 2)
    topk_w = jax.nn.softmax(topk_w, axis=-1)
    coef = jnp.zeros((N, E), jnp.float32)
    coef = coef.at[jnp.arange(N)[:, None], topk_i].set(topk_w)
    probs = jax.nn.softmax(logits, axis=-1)
    loss = E * jnp.sum(jnp.mean(probs, axis=0) ** 2)

    x_bf = x_flat.astype(jnp.bfloat16)
    wg_bf = w_gate.astype(jnp.bfloat16)
    wu_bf = w_up.astype(jnp.bfloat16)
    wd_bf = w_down.astype(jnp.bfloat16)

    BT = 1024
    NT = N // BT
    out = pl.pallas_call(
        _moe_kernel,
        grid=(NT, E),
        in_specs=[
            pl.BlockSpec((BT, D), lambda t, e: (t, 0)),
            pl.BlockSpec((BT, E), lambda t, e: (t, 0)),
            pl.BlockSpec((1, H, D), lambda t, e: (e, 0, 0)),
            pl.BlockSpec((1, H, D), lambda t, e: (e, 0, 0)),
            pl.BlockSpec((1, D, H), lambda t, e: (e, 0, 0)),
        ],
        out_specs=pl.BlockSpec((BT, D), lambda t, e: (t, 0)),
        out_shape=jax.ShapeDtypeStruct((N, D), jnp.float32),
    )(x_bf, coef, wg_bf, wu_bf, wd_bf)

    return out.reshape(Bb, Ss, D), loss, jnp.mean(probs, axis=0)


# speedup vs baseline: 1.6558x; 1.2796x over previous
import functools

import jax
import jax.numpy as jnp
from jax.experimental import pallas as pl
from jax.experimental.pallas import tpu as pltpu
from jax.experimental.pallas import tpu_sc as plsc

D = 1024
E = 8
H = 1792
N = 2048
NK = 2 * N
BT = 256
NB = NK // BT + E
SLOTS = NB * BT
NC = 2
NS = 16
NW = NC * NS
CH = NK // NW


def _cumsum_lanes(a):
    n = a.shape[1]
    k = 1
    while k < n:
        z = jnp.zeros((a.shape[0], k), a.dtype)
        a = a + jnp.concatenate([z, a[:, :n - k]], axis=1)
        k *= 2
    return a


def _excl_cumsum_sublanes(a):
    inc = a
    k = 1
    while k < a.shape[0]:
        z = jnp.zeros((k, a.shape[1]), a.dtype)
        inc = inc + jnp.concatenate([z, inc[:a.shape[0] - k]], axis=0)
        k *= 2
    return inc - a


def _router_kernel(lt_ref, pos_ref, w_ref, bexp_ref, psum_ref):
    lt = lt_ref[...]
    si = jax.lax.broadcasted_iota(jnp.int32, lt.shape, 0)
    m1 = jnp.max(lt, axis=0, keepdims=True)
    a1 = jnp.min(jnp.where(lt == m1, si, E), axis=0, keepdims=True)
    masked = jnp.where(si == a1, -jnp.inf, lt)
    m2 = jnp.max(masked, axis=0, keepdims=True)
    a2 = jnp.min(jnp.where(masked == m2, si, E), axis=0, keepdims=True)
    w1 = jax.nn.sigmoid(m1 - m2)

    o1 = (si == a1).astype(jnp.float32)
    o2 = (si == a2).astype(jnp.float32)
    c1 = _cumsum_lanes(o1)
    c2 = _cumsum_lanes(o2)
    cnt1 = c1[:, -1:]
    counts = cnt1 + c2[:, -1:]
    pc = jnp.ceil(counts * (1.0 / BT)) * BT
    pco = _excl_cumsum_sublanes(pc)
    rank1 = c1 - o1
    rank2 = cnt1 + c2 - o2
    pos1 = jnp.sum(o1 * (pco + rank1), axis=0, keepdims=True)
    pos2 = jnp.sum(o2 * (pco + rank2), axis=0, keepdims=True)
    pos_ref[...] = jnp.concatenate([pos1, pos2], axis=0).astype(jnp.int32)
    w_ref[...] = jnp.concatenate([w1, 1.0 - w1], axis=0)

    pend = pco + pc
    bstart = jax.lax.broadcasted_iota(
        jnp.int32, (E, NB), 1).astype(jnp.float32) * float(BT)
    bexp = jnp.sum((bstart >= pend).astype(jnp.float32),
                   axis=0, keepdims=True)
    bexp_ref[...] = jnp.minimum(bexp, E - 1.0).astype(jnp.int32)[0:1]

    ex = jnp.exp(lt - m1)
    p = ex / jnp.sum(ex, axis=0, keepdims=True)
    psum_ref[...] = jnp.sum(p, axis=1, keepdims=True)


@functools.lru_cache(maxsize=1)
def _make_sc_kernels():
    mesh = plsc.VectorSubcoreMesh(core_axis_name="c", subcore_axis_name="s",
                                  num_cores=NC, num_subcores=NS)

    HC = CH // 2

    @functools.partial(
        pl.kernel,
        out_type=jax.ShapeDtypeStruct((SLOTS, D), jnp.float32),
        mesh=mesh,
        scratch_types=[pltpu.VMEM((HC,), jnp.int32),
                       pltpu.VMEM((HC,), jnp.int32),
                       pltpu.VMEM((HC, D), jnp.float32),
                       pltpu.SemaphoreType.DMA],
    )
    def sc_dispatch(x_hbm, pos_hbm, xg_hbm, idx_a, idx_b, rows_v, sem):
        wid = jax.lax.axis_index("s") * NC + jax.lax.axis_index("c")
        base = wid * CH
        tok = jax.lax.rem(base, N)
        pltpu.sync_copy(pos_hbm.at[pl.ds(base, HC)], idx_a)
        pltpu.sync_copy(pos_hbm.at[pl.ds(base + HC, HC)], idx_b)
        pltpu.sync_copy(x_hbm.at[pl.ds(tok, HC)], rows_v)
        pltpu.async_copy(rows_v, xg_hbm.at[idx_a], sem).wait()
        pltpu.sync_copy(x_hbm.at[pl.ds(tok + HC, HC)], rows_v)
        pltpu.async_copy(rows_v, xg_hbm.at[idx_b], sem).wait()

    @functools.partial(
        pl.kernel,
        out_type=jax.ShapeDtypeStruct((NK, D), jnp.float32),
        mesh=mesh,
        scratch_types=[pltpu.VMEM((HC,), jnp.int32),
                       pltpu.VMEM((HC,), jnp.int32),
                       pltpu.VMEM((HC, D), jnp.float32),
                       pltpu.SemaphoreType.DMA],
    )
    def sc_combine(ygw_hbm, pos_hbm, g_hbm, idx_a, idx_b, rows_v, sem):
        wid = jax.lax.axis_index("s") * NC + jax.lax.axis_index("c")
        base = wid * CH
        pltpu.sync_copy(pos_hbm.at[pl.ds(base, HC)], idx_a)
        pltpu.sync_copy(pos_hbm.at[pl.ds(base + HC, HC)], idx_b)
        pltpu.async_copy(ygw_hbm.at[idx_a], rows_v, sem).wait()
        pltpu.sync_copy(rows_v, g_hbm.at[pl.ds(base, HC)])
        pltpu.async_copy(ygw_hbm.at[idx_b], rows_v, sem).wait()
        pltpu.sync_copy(rows_v, g_hbm.at[pl.ds(base + HC, HC)])

    return sc_dispatch, sc_combine


def _expert_kernel(bexp_ref, xg_ref, wg_ref, wu_ref, wd_ref, yg_ref):
    del bexp_ref
    xb = xg_ref[...].astype(jnp.bfloat16)
    g = jax.lax.dot_general(
        xb, wg_ref[0],
        dimension_numbers=(((1,), (1,)), ((), ())),
        preferred_element_type=jnp.float32)
    u = jax.lax.dot_general(
        xb, wu_ref[0],
        dimension_numbers=(((1,), (1,)), ((), ())),
        preferred_element_type=jnp.float32)
    h = (g * jax.nn.sigmoid(g) * u).astype(jnp.bfloat16)
    y = jax.lax.dot_general(
        h, wd_ref[0],
        dimension_numbers=(((1,), (1,)), ((), ())),
        preferred_element_type=jnp.float32)
    yg_ref[...] = y


def _add_kernel(g0_ref, g1_ref, w_ref, out_ref):
    w = w_ref[...]
    out_ref[...] = g0_ref[...] * w[:, 0:1] + g1_ref[...] * w[:, 1:2]


@jax.jit
def kernel(x, gate_w, w_gate, w_up, w_down):
    Bb, Ss, _ = x.shape
    x_flat = x.reshape(N, D)
    xb_g, gw_g = jax.lax.optimization_barrier(
        (x_flat.astype(jnp.bfloat16), gate_w.astype(jnp.bfloat16)))
    logits = jax.lax.optimization_barrier(
        xb_g @ gw_g.T).astype(jnp.float32)

    pos2, w2, bexp, psum = pl.pallas_call(
        _router_kernel,
        out_shape=[
            jax.ShapeDtypeStruct((2, N), jnp.int32),
            jax.ShapeDtypeStruct((2, N), jnp.float32),
            jax.ShapeDtypeStruct((1, NB), jnp.int32),
            jax.ShapeDtypeStruct((E, 1), jnp.float32),
        ],
    )(logits.T)

    pos_flat = pos2.reshape(NK)
    sc_dispatch, sc_combine = _make_sc_kernels()
    xg = sc_dispatch(x_flat, pos_flat)

    wg_bf = w_gate.astype(jnp.bfloat16)
    wu_bf = w_up.astype(jnp.bfloat16)
    wd_bf = w_down.astype(jnp.bfloat16)
    grid_spec = pltpu.PrefetchScalarGridSpec(
        num_scalar_prefetch=1,
        grid=(NB,),
        in_specs=[
            pl.BlockSpec((BT, D), lambda b, t: (b, 0)),
            pl.BlockSpec((1, H, D), lambda b, t: (t[b], 0, 0)),
            pl.BlockSpec((1, H, D), lambda b, t: (t[b], 0, 0)),
            pl.BlockSpec((1, D, H), lambda b, t: (t[b], 0, 0)),
        ],
        out_specs=pl.BlockSpec((BT, D), lambda b, t: (b, 0)),
    )
    yg = pl.pallas_call(
        _expert_kernel,
        grid_spec=grid_spec,
        out_shape=jax.ShapeDtypeStruct((SLOTS, D), jnp.float32),
    )(bexp.reshape(NB), xg, wg_bf, wu_bf, wd_bf)

    g = sc_combine(yg, pos_flat)

    BTA = 512
    out = pl.pallas_call(
        _add_kernel,
        grid=(N // BTA,),
        in_specs=[
            pl.BlockSpec((BTA, D), lambda t: (t, 0)),
            pl.BlockSpec((BTA, D), lambda t: (t + N // BTA, 0)),
            pl.BlockSpec((BTA, 2), lambda t: (t, 0)),
        ],
        out_specs=pl.BlockSpec((BTA, D), lambda t: (t, 0)),
        out_shape=jax.ShapeDtypeStruct((N, D), jnp.float32),
    )(g, g, w2.T)

    probs_mean = psum[:, 0] / N
    loss = E * jnp.sum(probs_mean ** 2)
    return out.reshape(Bb, Ss, D), loss, probs_mean
